# Initial kernel scaffold; baseline (speedup 1.0000x reference)
import jax
import jax.numpy as jnp
from jax.experimental import pallas as pl


def kernel(token_ids, table):
    raise NotImplementedError("write your pallas kernel here")


# speedup vs baseline: 2.4880x; 2.4880x over previous
import functools

import jax
import jax.numpy as jnp
from jax import lax
from jax.experimental import pallas as pl
from jax.experimental.pallas import tpu as pltpu
from jax.experimental.pallas import tpu_sc as plsc

NUM_EMBEDDINGS = 1000000
D = 64
B = 16384
S = 50

NC = 2
NS = 16
NW = NC * NS
B_PER_W = B // NW
CHUNK_B = 16
N_CHUNKS = B_PER_W // CHUNK_B
IDX_PER_CHUNK = CHUNK_B * S
GATHER_W = 80
N_GATHERS = IDX_PER_CHUNK // GATHER_W
L = 16


def _body(tok_hbm, table_hbm, out_hbm, idx_v, rows_v, out_v, sem):
    wid = lax.axis_index("s") * NC + lax.axis_index("c")

    @pl.loop(0, N_CHUNKS)
    def _chunk(c):
        chunk_b = wid * B_PER_W + c * CHUNK_B
        pltpu.sync_copy(tok_hbm.at[pl.ds(chunk_b * S, IDX_PER_CHUNK)], idx_v)

        copies = []
        for j in range(N_GATHERS):
            copies.append(
                pltpu.async_copy(
                    table_hbm.at[idx_v.at[pl.ds(j * GATHER_W, GATHER_W)]],
                    rows_v.at[pl.ds(j * GATHER_W, GATHER_W)],
                    sem,
                )
            )
        for cp in copies:
            cp.wait()

        @pl.loop(0, CHUNK_B)
        def _row(b):
            base = b * S
            accs = [rows_v[base, pl.ds(d * L, L)] for d in range(D // L)]
            for s in range(1, S):
                for d in range(D // L):
                    accs[d] = accs[d] + rows_v[base + s, pl.ds(d * L, L)]
            for d in range(D // L):
                out_v[b, pl.ds(d * L, L)] = accs[d]

        pltpu.sync_copy(out_v, out_hbm.at[pl.ds(chunk_b, CHUNK_B)])


@jax.jit
def kernel(token_ids, table):
    tok1d = token_ids.reshape(B * S).astype(jnp.int32)
    mesh = plsc.VectorSubcoreMesh(core_axis_name="c", subcore_axis_name="s")
    k = pl.kernel(
        _body,
        out_type=jax.ShapeDtypeStruct((B, D), jnp.float32),
        mesh=mesh,
        scratch_types=[
            pltpu.VMEM((IDX_PER_CHUNK,), jnp.int32),
            pltpu.VMEM((IDX_PER_CHUNK, D), jnp.float32),
            pltpu.VMEM((CHUNK_B, D), jnp.float32),
            pltpu.SemaphoreType.DMA,
        ],
        compiler_params=pltpu.CompilerParams(use_tc_tiling_on_sc=False),
    )
    return k(tok1d, table)


# speedup vs baseline: 2.7784x; 1.1167x over previous
import jax
import jax.numpy as jnp
from jax import lax
from jax.experimental import pallas as pl
from jax.experimental.pallas import tpu as pltpu
from jax.experimental.pallas import tpu_sc as plsc

NUM_EMBEDDINGS = 1000000
D = 64
B = 16384
S = 50

NC = 2
NS = 16
NW = NC * NS
B_PER_W = B // NW
CHUNK_B = 8
N_CHUNKS = B_PER_W // CHUNK_B
IDX_PER_CHUNK = CHUNK_B * S
GATHER_W = 80
N_GATHERS = IDX_PER_CHUNK // GATHER_W
L = 16


def _body(tok_hbm, table_hbm, out_hbm, idx_all, rows_v, out_all, gsem, osem):
    wid = lax.axis_index("s") * NC + lax.axis_index("c")
    tile_idx0 = wid * B_PER_W * S

    pltpu.sync_copy(tok_hbm.at[pl.ds(tile_idx0, B_PER_W * S)], idx_all)

    def fire(cc, p):
        for j in range(N_GATHERS):
            pltpu.async_copy(
                table_hbm.at[
                    idx_all.at[pl.ds(cc * IDX_PER_CHUNK + j * GATHER_W, GATHER_W)]
                ],
                rows_v.at[p, pl.ds(j * GATHER_W, GATHER_W)],
                gsem.at[p],
            )

    def drain(p):
        pltpu.make_async_copy(
            table_hbm.at[pl.ds(0, IDX_PER_CHUNK)], rows_v.at[p], gsem.at[p]
        ).wait()

    fire(0, 0)

    @pl.loop(0, N_CHUNKS, step=2)
    def _chunks(c):
        for par in range(2):
            cc = c + par

            @pl.when(cc + 1 < N_CHUNKS)
            def _():
                fire(cc + 1, 1 - par)

            drain(par)

            @pl.loop(0, CHUNK_B)
            def _row(b):
                base = b * S
                accs = [rows_v[par, base, pl.ds(d * L, L)] for d in range(D // L)]
                for s in range(1, S):
                    for d in range(D // L):
                        accs[d] = accs[d] + rows_v[par, base + s, pl.ds(d * L, L)]
                orow = cc * CHUNK_B + b
                for d in range(D // L):
                    out_all[orow, pl.ds(d * L, L)] = accs[d]

    pltpu.async_copy(out_all, out_hbm.at[pl.ds(wid * B_PER_W, B_PER_W)], osem).wait()


@jax.jit
def kernel(token_ids, table):
    tok1d = token_ids.reshape(B * S).astype(jnp.int32)
    mesh = plsc.VectorSubcoreMesh(core_axis_name="c", subcore_axis_name="s")
    k = pl.kernel(
        _body,
        out_type=jax.ShapeDtypeStruct((B, D), jnp.float32),
        mesh=mesh,
        scratch_types=[
            pltpu.VMEM((B_PER_W * S,), jnp.int32),
            pltpu.VMEM((2, IDX_PER_CHUNK, D), jnp.float32),
            pltpu.VMEM((B_PER_W, D), jnp.float32),
            pltpu.SemaphoreType.DMA((2,)),
            pltpu.SemaphoreType.DMA,
        ],
        compiler_params=pltpu.CompilerParams(use_tc_tiling_on_sc=False),
    )
    return k(tok1d, table)
